# Initial kernel scaffold; baseline (speedup 1.0000x reference)
import jax
import jax.numpy as jnp
from jax.experimental import pallas as pl


def kernel(marg_i, cav_ij, C, field_i, edge_src, edge_dst, indice_ij):
    raise NotImplementedError("write your pallas kernel here")


# speedup vs baseline: 2.0283x; 2.0283x over previous
import jax
import jax.numpy as jnp
from jax import lax
from jax.experimental import pallas as pl
from jax.experimental.pallas import tpu as pltpu
from jax.experimental.pallas import tpu_sc as plsc

EPS = 1e-10
N_NODES = 10000
N_EDGES = 320000
Q = 128

NC = 2
NS = 16
NW = NC * NS
EPT = N_EDGES // NW
CH = 80
NCHUNK = EPT // CH
NPAD = 10240
NPT = NPAD // NS
ZH = 128
NZH = NPT // ZH
L = 16


def _mmlog_body(cav_ref, c_ref, out_ref):
    acc = jnp.dot(cav_ref[...], c_ref[...], preferred_element_type=jnp.float32)
    out_ref[...] = jnp.log(acc + EPS)


def _mmlog(cav_ij, C):
    bm = 1280
    return pl.pallas_call(
        _mmlog_body,
        grid=(N_EDGES // bm,),
        in_specs=[
            pl.BlockSpec((bm, Q), lambda i: (i, 0)),
            pl.BlockSpec((Q, Q), lambda i: (0, 0)),
        ],
        out_specs=pl.BlockSpec((bm, Q), lambda i: (i, 0)),
        out_shape=jax.ShapeDtypeStruct((N_EDGES, Q), jnp.float32),
    )(cav_ij, C)


def _scatter_body(temp2_hbm, dst_hbm, out0_hbm, out1_hbm,
                  acc_shared, rows_v, idx_v, stage_v):
    c = lax.axis_index("c")
    s = lax.axis_index("s")
    wid = c * NS + s

    zf = jnp.zeros((L,), jnp.float32)

    def _zb(r, _):
        for k in range(Q // L):
            stage_v[r, pl.ds(k * L, L)] = zf
        return 0
    lax.fori_loop(0, ZH, _zb, 0)
    for h in range(NZH):
        pltpu.sync_copy(stage_v, acc_shared.at[pl.ds(s * NPT + h * ZH, ZH)])
    plsc.subcore_barrier()

    def _chunk(i, _):
        pltpu.sync_copy(dst_hbm.at[pl.ds(wid * EPT + i * CH, CH)], idx_v)
        pltpu.sync_copy(temp2_hbm.at[pl.ds(wid * EPT + i * CH, CH)], rows_v)
        pltpu.sync_copy(rows_v, acc_shared.at[idx_v], add=True)
        return 0
    lax.fori_loop(0, NCHUNK, _chunk, 0)
    plsc.subcore_barrier()

    for h in range(NZH):
        base = s * NPT + h * ZH
        pltpu.sync_copy(acc_shared.at[pl.ds(base, ZH)], stage_v)

        @pl.when(c == 0)
        def _():
            pltpu.sync_copy(stage_v, out0_hbm.at[pl.ds(base, ZH)])

        @pl.when(c == 1)
        def _():
            pltpu.sync_copy(stage_v, out1_hbm.at[pl.ds(base, ZH)])


def _segment_sum_sc(temp2, dst_idx):
    f = pl.kernel(
        _scatter_body,
        out_type=(jax.ShapeDtypeStruct((NPAD, Q), jnp.float32),
                  jax.ShapeDtypeStruct((NPAD, Q), jnp.float32)),
        mesh=plsc.VectorSubcoreMesh(core_axis_name="c", subcore_axis_name="s"),
        scratch_types=[
            pltpu.VMEM_SHARED((NPAD, Q), jnp.float32),
            pltpu.VMEM((CH, Q), jnp.float32),
            pltpu.VMEM((CH,), jnp.int32),
            pltpu.VMEM((ZH, Q), jnp.float32),
        ],
    )
    return f(temp2, dst_idx)


def _combine_body(f_ref, p0_ref, p1_ref, out_ref):
    out_ref[...] = f_ref[...] + p0_ref[...] + p1_ref[...]


def _combine(field_i, p0, p1):
    bn = 1000
    spec = pl.BlockSpec((bn, Q), lambda i: (i, 0))
    return pl.pallas_call(
        _combine_body,
        grid=(N_NODES // bn,),
        in_specs=[spec, spec, spec],
        out_specs=spec,
        out_shape=jax.ShapeDtypeStruct((N_NODES, Q), jnp.float32),
    )(field_i, p0, p1)


def _gather_body(marg_hbm, temp2_hbm, src_hbm, ind_hbm, out_hbm,
                 a_v, b_v, o_v, isrc_v, iind_v, sem_a, sem_b):
    c = lax.axis_index("c")
    s = lax.axis_index("s")
    wid = c * NS + s

    def _chunk(i, _):
        base = wid * EPT + i * CH
        pltpu.sync_copy(src_hbm.at[pl.ds(base, CH)], isrc_v)
        pltpu.sync_copy(ind_hbm.at[pl.ds(base, CH)], iind_v)
        cp_a = pltpu.async_copy(marg_hbm.at[isrc_v], a_v, sem_a)
        cp_b = pltpu.async_copy(temp2_hbm.at[iind_v], b_v, sem_b)
        cp_a.wait()
        cp_b.wait()

        def _row(r, _):
            for k in range(Q // L):
                sl = pl.ds(k * L, L)
                o_v[r, sl] = a_v[r, sl] - b_v[r, sl]
            return 0
        lax.fori_loop(0, CH, _row, 0)
        pltpu.sync_copy(o_v, out_hbm.at[pl.ds(base, CH)])
        return 0
    lax.fori_loop(0, NCHUNK, _chunk, 0)


def _gather_sub_sc(marg, temp2, src_idx, ind_idx):
    f = pl.kernel(
        _gather_body,
        out_type=jax.ShapeDtypeStruct((N_EDGES, Q), jnp.float32),
        mesh=plsc.VectorSubcoreMesh(core_axis_name="c", subcore_axis_name="s"),
        scratch_types=[
            pltpu.VMEM((CH, Q), jnp.float32),
            pltpu.VMEM((CH, Q), jnp.float32),
            pltpu.VMEM((CH, Q), jnp.float32),
            pltpu.VMEM((CH,), jnp.int32),
            pltpu.VMEM((CH,), jnp.int32),
            pltpu.SemaphoreType.DMA,
            pltpu.SemaphoreType.DMA,
        ],
    )
    return f(marg, temp2, src_idx, ind_idx)


def _softmax_body(x_ref, out_ref):
    x = x_ref[...]
    m = jnp.max(x, axis=1, keepdims=True)
    e = jnp.exp(x - m)
    out_ref[...] = e / jnp.sum(e, axis=1, keepdims=True)


def _softmax(logits):
    bm = 1280
    spec = pl.BlockSpec((bm, Q), lambda i: (i, 0))
    return pl.pallas_call(
        _softmax_body,
        grid=(N_EDGES // bm,),
        in_specs=[spec],
        out_specs=spec,
        out_shape=jax.ShapeDtypeStruct((N_EDGES, Q), jnp.float32),
    )(logits)


def kernel(marg_i, cav_ij, C, field_i, edge_src, edge_dst, indice_ij):
    del marg_i
    temp2 = _mmlog(cav_ij, C)
    p0, p1 = _segment_sum_sc(temp2, edge_dst.astype(jnp.int32))
    marg = _combine(field_i, p0, p1)
    logits = _gather_sub_sc(marg, temp2, edge_src.astype(jnp.int32),
                            indice_ij.astype(jnp.int32))
    cav = _softmax(logits)
    return (marg, cav)


# speedup vs baseline: 3.0580x; 1.5077x over previous
import jax
import jax.numpy as jnp
from jax import lax
from jax.experimental import pallas as pl
from jax.experimental.pallas import tpu as pltpu
from jax.experimental.pallas import tpu_sc as plsc

EPS = 1e-10
N_NODES = 10000
N_EDGES = 320000
Q = 128

NC = 2
NS = 16
NW = NC * NS
EPT = N_EDGES // NW
CH = 80
NCHUNK = EPT // CH
NPAD = 10240
NPT = NPAD // NS
ZH = 128
NZH = NPT // ZH
L = 16


def _mmlog_body(cav_ref, c_ref, out_ref):
    acc = jnp.dot(cav_ref[...], c_ref[...], preferred_element_type=jnp.float32)
    out_ref[...] = jnp.log(acc + EPS)


def _mmlog(cav_ij, C):
    bm = 1280
    return pl.pallas_call(
        _mmlog_body,
        grid=(N_EDGES // bm,),
        in_specs=[
            pl.BlockSpec((bm, Q), lambda i: (i, 0)),
            pl.BlockSpec((Q, Q), lambda i: (0, 0)),
        ],
        out_specs=pl.BlockSpec((bm, Q), lambda i: (i, 0)),
        out_shape=jax.ShapeDtypeStruct((N_EDGES, Q), jnp.float32),
    )(cav_ij, C)


def _scatter_body(temp2_hbm, dst_hbm, out0_hbm, out1_hbm,
                  acc_shared, rows0_v, rows1_v, idx0_v, idx1_v, stage_v,
                  sl0, sl1):
    c = lax.axis_index("c")
    s = lax.axis_index("s")
    wid = c * NS + s
    rows = (rows0_v, rows1_v)
    idx = (idx0_v, idx1_v)
    sl = (sl0, sl1)

    zf = jnp.zeros((L,), jnp.float32)

    def _zb(r, _):
        for k in range(Q // L):
            stage_v[r, pl.ds(k * L, L)] = zf
        return 0
    lax.fori_loop(0, ZH, _zb, 0)
    for h in range(NZH):
        pltpu.sync_copy(stage_v, acc_shared.at[pl.ds(s * NPT + h * ZH, ZH)])
    plsc.subcore_barrier()

    def _load_start(i, p):
        base = wid * EPT + i * CH
        pltpu.async_copy(dst_hbm.at[pl.ds(base, CH)], idx[p], sl[p])
        pltpu.async_copy(temp2_hbm.at[pl.ds(base, CH)], rows[p], sl[p])

    def _load_wait(p):
        pltpu.make_async_copy(dst_hbm.at[pl.ds(0, CH)], idx[p], sl[p]).wait()
        pltpu.make_async_copy(temp2_hbm.at[pl.ds(0, CH)], rows[p], sl[p]).wait()

    _load_start(0, 0)

    def _pair(j, _):
        i0 = 2 * j
        for p in (0, 1):
            i = i0 + p
            _load_start(i + 1, 1 - p)
            _load_wait(p)
            pltpu.sync_copy(rows[p], acc_shared.at[idx[p]], add=True)
        return 0
    lax.fori_loop(0, (NCHUNK - 1) // 2, _pair, 0)
    _load_wait(0)
    pltpu.sync_copy(rows[0], acc_shared.at[idx[0]], add=True)
    plsc.subcore_barrier()

    for h in range(NZH):
        base = s * NPT + h * ZH
        pltpu.sync_copy(acc_shared.at[pl.ds(base, ZH)], stage_v)

        @pl.when(c == 0)
        def _():
            pltpu.sync_copy(stage_v, out0_hbm.at[pl.ds(base, ZH)])

        @pl.when(c == 1)
        def _():
            pltpu.sync_copy(stage_v, out1_hbm.at[pl.ds(base, ZH)])


def _segment_sum_sc(temp2, dst_idx):
    f = pl.kernel(
        _scatter_body,
        out_type=(jax.ShapeDtypeStruct((NPAD, Q), jnp.float32),
                  jax.ShapeDtypeStruct((NPAD, Q), jnp.float32)),
        mesh=plsc.VectorSubcoreMesh(core_axis_name="c", subcore_axis_name="s"),
        scratch_types=[
            pltpu.VMEM_SHARED((NPAD, Q), jnp.float32),
            pltpu.VMEM((CH, Q), jnp.float32),
            pltpu.VMEM((CH, Q), jnp.float32),
            pltpu.VMEM((CH,), jnp.int32),
            pltpu.VMEM((CH,), jnp.int32),
            pltpu.VMEM((ZH, Q), jnp.float32),
            pltpu.SemaphoreType.DMA,
            pltpu.SemaphoreType.DMA,
        ],
    )
    return f(temp2, dst_idx)


def _combine_body(f_ref, p0_ref, p1_ref, out_ref):
    out_ref[...] = f_ref[...] + p0_ref[...] + p1_ref[...]


def _combine(field_i, p0, p1):
    bn = 1000
    spec = pl.BlockSpec((bn, Q), lambda i: (i, 0))
    return pl.pallas_call(
        _combine_body,
        grid=(N_NODES // bn,),
        in_specs=[spec, spec, spec],
        out_specs=spec,
        out_shape=jax.ShapeDtypeStruct((N_NODES, Q), jnp.float32),
    )(field_i, p0, p1)


def _gather_body(marg_hbm, temp2_hbm, src_hbm, ind_hbm, out_hbm,
                 a0_v, a1_v, b0_v, b1_v, o0_v, o1_v,
                 isrc0_v, isrc1_v, iind0_v, iind1_v,
                 sg0, sg1, si0, si1, so0, so1):
    c = lax.axis_index("c")
    s = lax.axis_index("s")
    wid = c * NS + s
    ebase = wid * EPT
    a = (a0_v, a1_v)
    b = (b0_v, b1_v)
    o = (o0_v, o1_v)
    isrc = (isrc0_v, isrc1_v)
    iind = (iind0_v, iind1_v)
    sg = (sg0, sg1)
    si = (si0, si1)
    so = (so0, so1)

    def _idx_start(i, p):
        base = ebase + i * CH
        pltpu.async_copy(src_hbm.at[pl.ds(base, CH)], isrc[p], si[p])
        pltpu.async_copy(ind_hbm.at[pl.ds(base, CH)], iind[p], si[p])

    def _idx_wait(p):
        pltpu.make_async_copy(src_hbm.at[pl.ds(0, CH)], isrc[p], si[p]).wait()
        pltpu.make_async_copy(ind_hbm.at[pl.ds(0, CH)], iind[p], si[p]).wait()

    def _gather_start(p):
        pltpu.async_copy(marg_hbm.at[isrc[p]], a[p], sg[p])
        pltpu.async_copy(temp2_hbm.at[iind[p]], b[p], sg[p])

    def _gather_wait(p):
        pltpu.make_async_copy(marg_hbm.at[pl.ds(0, CH)], a[p], sg[p]).wait()
        pltpu.make_async_copy(temp2_hbm.at[pl.ds(0, CH)], b[p], sg[p]).wait()

    def _compute(p):
        def _row(r, _):
            for k in range(Q // L):
                sl = pl.ds(k * L, L)
                o[p][r, sl] = a[p][r, sl] - b[p][r, sl]
            return 0
        lax.fori_loop(0, CH, _row, 0)

    def _out_start(i, p):
        pltpu.async_copy(o[p], out_hbm.at[pl.ds(ebase + i * CH, CH)], so[p])

    def _out_wait(p):
        pltpu.make_async_copy(o[p], out_hbm.at[pl.ds(0, CH)], so[p]).wait()

    _idx_start(0, 0)
    _idx_wait(0)
    _gather_start(0)
    _idx_start(1, 1)

    def _pair(j, _):
        i0 = 2 * j
        for p in (0, 1):
            i = i0 + p
            _idx_wait(1 - p)
            _gather_start(1 - p)
            _gather_wait(p)

            @pl.when(i + 2 < NCHUNK)
            def _():
                _idx_start(i + 2, p)

            @pl.when(j > 0)
            def _():
                _out_wait(p)
            _compute(p)
            _out_start(i, p)
        return 0
    lax.fori_loop(0, (NCHUNK - 1) // 2, _pair, 0)
    _gather_wait(0)
    _out_wait(0)
    _compute(0)
    _out_start(NCHUNK - 1, 0)
    _out_wait(1)
    _out_wait(0)


def _gather_sub_sc(marg, temp2, src_idx, ind_idx):
    f = pl.kernel(
        _gather_body,
        out_type=jax.ShapeDtypeStruct((N_EDGES, Q), jnp.float32),
        mesh=plsc.VectorSubcoreMesh(core_axis_name="c", subcore_axis_name="s"),
        scratch_types=(
            [pltpu.VMEM((CH, Q), jnp.float32)] * 6
            + [pltpu.VMEM((CH,), jnp.int32)] * 4
            + [pltpu.SemaphoreType.DMA] * 6
        ),
    )
    return f(marg, temp2, src_idx, ind_idx)


def _softmax_body(x_ref, out_ref):
    x = x_ref[...]
    m = jnp.max(x, axis=1, keepdims=True)
    e = jnp.exp(x - m)
    r = 1.0 / jnp.sum(e, axis=1, keepdims=True)
    out_ref[...] = e * r


def _softmax(logits):
    bm = 1280
    spec = pl.BlockSpec((bm, Q), lambda i: (i, 0))
    return pl.pallas_call(
        _softmax_body,
        grid=(N_EDGES // bm,),
        in_specs=[spec],
        out_specs=spec,
        out_shape=jax.ShapeDtypeStruct((N_EDGES, Q), jnp.float32),
    )(logits)


def kernel(marg_i, cav_ij, C, field_i, edge_src, edge_dst, indice_ij):
    del marg_i
    temp2 = _mmlog(cav_ij, C)
    p0, p1 = _segment_sum_sc(temp2, edge_dst.astype(jnp.int32))
    marg = _combine(field_i, p0, p1)
    logits = _gather_sub_sc(marg, temp2, edge_src.astype(jnp.int32),
                            indice_ij.astype(jnp.int32))
    cav = _softmax(logits)
    return (marg, cav)
